# Initial kernel scaffold; baseline (speedup 1.0000x reference)
import jax
import jax.numpy as jnp
from jax.experimental import pallas as pl


def kernel(x, node_ids, W_word, W_graph):
    raise NotImplementedError("write your pallas kernel here")


# speedup vs baseline: 3.8587x; 3.8587x over previous
import functools

import jax
import jax.numpy as jnp
from jax import lax
from jax.experimental import pallas as pl
from jax.experimental.pallas import tpu as pltpu
from jax.experimental.pallas import tpu_sc as plsc

D = 128
NW = 32
CW = 128
CN = 80


def _make_embed(BW: int, BN: int):
    wpw = BW // NW
    npw = BN // NW
    wch = wpw // CW
    nch = npw // CN
    assert wpw % CW == 0 and npw % CN == 0

    mesh = plsc.VectorSubcoreMesh(core_axis_name="c", subcore_axis_name="s")

    @functools.partial(
        pl.kernel,
        mesh=mesh,
        out_type=(
            jax.ShapeDtypeStruct((BW, D), jnp.float32),
            jax.ShapeDtypeStruct((BN, D), jnp.float32),
        ),
        scratch_types=[
            pltpu.VMEM((wpw,), jnp.int32),
            pltpu.VMEM((npw,), jnp.int32),
            pltpu.VMEM((CW, D), jnp.float32),
            pltpu.SemaphoreType.DMA,
            pltpu.SemaphoreType.DMA,
        ],
    )
    def embed(x_hbm, nid_hbm, ww_hbm, wg_hbm, out_w, out_n,
              idxw_v, idxn_v, buf, gsem, osem):
        wid = lax.axis_index("s") * 2 + lax.axis_index("c")
        wbase = wid * wpw
        nbase = wid * npw

        pltpu.sync_copy(x_hbm.at[pl.ds(wbase, wpw)], idxw_v)
        pltpu.sync_copy(nid_hbm.at[pl.ds(nbase, npw)], idxn_v)

        def wchunk(j, carry):
            idx = idxw_v.at[pl.ds(j * CW, CW)]
            pltpu.async_copy(ww_hbm.at[idx], buf, gsem).wait()
            pltpu.async_copy(
                buf, out_w.at[pl.ds(wbase + j * CW, CW)], osem).wait()
            return carry

        lax.fori_loop(0, wch, wchunk, 0)

        nbuf = buf.at[pl.ds(0, CN)]

        def nchunk(j, carry):
            idx = idxn_v.at[pl.ds(j * CN, CN)]
            pltpu.async_copy(wg_hbm.at[idx], nbuf, gsem).wait()
            pltpu.async_copy(
                nbuf, out_n.at[pl.ds(nbase + j * CN, CN)], osem).wait()
            return carry

        lax.fori_loop(0, nch, nchunk, 0)

    return embed


def kernel(x, node_ids, W_word, W_graph):
    B, S = x.shape
    _, N = node_ids.shape
    BW = B * S
    BN = B * N
    x1 = x.reshape(BW)
    n1 = node_ids.reshape(BN)
    out_w, out_n = _make_embed(BW, BN)(x1, n1, W_word, W_graph)
    return out_w.reshape(B, S, D), out_n.reshape(B, N, D)


# speedup vs baseline: 4.6808x; 1.2130x over previous
import functools

import jax
import jax.numpy as jnp
from jax import lax
from jax.experimental import pallas as pl
from jax.experimental.pallas import tpu as pltpu
from jax.experimental.pallas import tpu_sc as plsc

D = 128
NW = 32
CW = 128
CN = 80


def _make_embed(BW: int, BN: int):
    wpw = BW // NW
    npw = BN // NW
    wch = wpw // CW
    nch = npw // CN
    assert wpw % CW == 0 and npw % CN == 0
    assert wch % 2 == 0 and nch % 2 == 0

    mesh = plsc.VectorSubcoreMesh(core_axis_name="c", subcore_axis_name="s")

    @functools.partial(
        pl.kernel,
        mesh=mesh,
        out_type=(
            jax.ShapeDtypeStruct((BW, D), jnp.float32),
            jax.ShapeDtypeStruct((BN, D), jnp.float32),
        ),
        scratch_types=[
            pltpu.VMEM((wpw,), jnp.int32),
            pltpu.VMEM((npw,), jnp.int32),
            pltpu.VMEM((CW, D), jnp.float32),
            pltpu.VMEM((CW, D), jnp.float32),
            pltpu.SemaphoreType.DMA,
            pltpu.SemaphoreType.DMA,
            pltpu.SemaphoreType.DMA,
            pltpu.SemaphoreType.DMA,
        ],
    )
    def embed(x_hbm, nid_hbm, ww_hbm, wg_hbm, out_w, out_n,
              idxw_v, idxn_v, buf0, buf1, gsem0, gsem1, osem0, osem1):
        wid = lax.axis_index("s") * 2 + lax.axis_index("c")
        wbase = wid * wpw
        nbase = wid * npw

        pltpu.sync_copy(x_hbm.at[pl.ds(wbase, wpw)], idxw_v)
        pltpu.sync_copy(nid_hbm.at[pl.ds(nbase, npw)], idxn_v)

        def run_table(tbl, idx_v, out, base, C, nchunks):
            bufs = (buf0.at[pl.ds(0, C)], buf1.at[pl.ds(0, C)])
            gsems = (gsem0, gsem1)
            osems = (osem0, osem1)

            def gather_desc(j, b):
                idx = idx_v.at[pl.ds(j * C, C)]
                return pltpu.make_async_copy(tbl.at[idx], bufs[b], gsems[b])

            def ocopy_desc(j, b):
                dst = out.at[pl.ds(base + j * C, C)]
                return pltpu.make_async_copy(bufs[b], dst, osems[b])

            gather_desc(0, 0).start()
            gather_desc(1, 1).start()

            def body(i, carry):
                j0 = i * 2
                for b in range(2):
                    gather_desc(j0 + b, b).wait()
                    ocopy_desc(j0 + b, b).start()
                for b in range(2):
                    ocopy_desc(j0 + b, b).wait()

                    @pl.when(j0 + 2 + b < nchunks)
                    def _():
                        gather_desc(j0 + 2 + b, b).start()
                return carry

            lax.fori_loop(0, nchunks // 2, body, 0)

        run_table(ww_hbm, idxw_v, out_w, wbase, CW, wch)
        run_table(wg_hbm, idxn_v, out_n, nbase, CN, nch)

    return embed


def kernel(x, node_ids, W_word, W_graph):
    B, S = x.shape
    _, N = node_ids.shape
    BW = B * S
    BN = B * N
    x1 = x.reshape(BW)
    n1 = node_ids.reshape(BN)
    out_w, out_n = _make_embed(BW, BN)(x1, n1, W_word, W_graph)
    return out_w.reshape(B, S, D), out_n.reshape(B, N, D)


# speedup vs baseline: 4.9454x; 1.0565x over previous
import functools

import jax
import jax.numpy as jnp
from jax import lax
from jax.experimental import pallas as pl
from jax.experimental.pallas import tpu as pltpu
from jax.experimental.pallas import tpu_sc as plsc

D = 128
NW = 32
CW = 128
CN = 80


def _make_embed(BW: int, BN: int):
    wpw = BW // NW
    npw = BN // NW
    wch = wpw // CW
    nch = npw // CN
    assert wpw % CW == 0 and npw % CN == 0
    NBUF = 4

    mesh = plsc.VectorSubcoreMesh(core_axis_name="c", subcore_axis_name="s")

    @functools.partial(
        pl.kernel,
        mesh=mesh,
        out_type=(
            jax.ShapeDtypeStruct((BW, D), jnp.float32),
            jax.ShapeDtypeStruct((BN, D), jnp.float32),
        ),
        scratch_types=[
            pltpu.VMEM((wpw,), jnp.int32),
            pltpu.VMEM((npw,), jnp.int32),
            pltpu.VMEM((NBUF, CW, D), jnp.float32),
            pltpu.SemaphoreType.DMA,
            pltpu.SemaphoreType.DMA,
        ]
        + [pltpu.SemaphoreType.DMA] * (2 * NBUF),
    )
    def embed(x_hbm, nid_hbm, ww_hbm, wg_hbm, out_w, out_n,
              idxw_v, idxn_v, bufs_v, isem0, isem1, *sems):
        gsems = sems[:NBUF]
        osems = sems[NBUF:]
        wid = lax.axis_index("s") * 2 + lax.axis_index("c")
        wbase = wid * wpw
        nbase = wid * npw

        widx_cp = pltpu.make_async_copy(
            x_hbm.at[pl.ds(wbase, wpw)], idxw_v, isem0)
        nidx_cp = pltpu.make_async_copy(
            nid_hbm.at[pl.ds(nbase, npw)], idxn_v, isem1)
        widx_cp.start()
        nidx_cp.start()

        def run_table(tbl, idx_v, out, base, C, nchunks):
            bufs = tuple(bufs_v.at[b, pl.ds(0, C)] for b in range(NBUF))

            def gather_desc(j, b):
                idx = idx_v.at[pl.ds(j * C, C)]
                return pltpu.make_async_copy(tbl.at[idx], bufs[b], gsems[b])

            def ocopy_desc(j, b):
                dst = out.at[pl.ds(base + j * C, C)]
                return pltpu.make_async_copy(bufs[b], dst, osems[b])

            for b in range(min(NBUF, nchunks)):
                gather_desc(b, b).start()

            def body(i, carry):
                j0 = i * NBUF
                for b in range(NBUF):
                    j = j0 + b

                    @pl.when(j < nchunks)
                    def _():
                        gather_desc(j, b).wait()
                        ocopy_desc(j, b).start()
                for b in range(NBUF):
                    j = j0 + b

                    @pl.when(j < nchunks)
                    def _():
                        ocopy_desc(j, b).wait()

                        @pl.when(j + NBUF < nchunks)
                        def _():
                            gather_desc(j + NBUF, b).start()
                return carry

            lax.fori_loop(0, (nchunks + NBUF - 1) // NBUF, body, 0)

        widx_cp.wait()
        run_table(ww_hbm, idxw_v, out_w, wbase, CW, wch)
        nidx_cp.wait()
        run_table(wg_hbm, idxn_v, out_n, nbase, CN, nch)

    return embed


def kernel(x, node_ids, W_word, W_graph):
    B, S = x.shape
    _, N = node_ids.shape
    BW = B * S
    BN = B * N
    x1 = x.reshape(BW)
    n1 = node_ids.reshape(BN)
    out_w, out_n = _make_embed(BW, BN)(x1, n1, W_word, W_graph)
    return out_w.reshape(B, S, D), out_n.reshape(B, N, D)
